# baseline (speedup 1.0000x reference)
import functools

import jax
import jax.numpy as jnp
from jax import lax
from jax.experimental import pallas as pl
from jax.experimental.pallas import tpu as pltpu
from jax.experimental.pallas import tpu_sc as plsc

N = 10000
E = 160000
D = 256
HALF = 128
EPS = 1e-5

NC, NS = 2, 16
K = 48
CHUNKS = 216
KC = 128
CHUNKS_C = 81
EP = K * CHUNKS
E_PAD = EP * NS
NROWS = 10112

_f32 = jnp.float32
_i32 = jnp.int32


def _sc_run_dir(c, s, emb2, srcs, dsts, out_s,
                src_v, dst_v, rows_v, acc,
                semg0, semg1, semg2, semg3, sems0, sems1, sems2, sems3):
    w = c * NS + s
    semg = (semg0, semg1, semg2, semg3)
    sems = (sems0, sems1, sems2, sems3)

    pltpu.sync_copy(srcs.at[w], src_v)
    pltpu.sync_copy(dsts.at[s], dst_v)

    pltpu.make_async_copy(emb2.at[src_v.at[0]], rows_v.at[0], semg0).start()

    r3 = rows_v.at[3]

    def _zrow(i, _):
        for j in range(8):
            rows_v[3, i, pl.ds(j * 16, 16)] = jnp.zeros((16,), _f32)
        return 0
    lax.fori_loop(0, K, _zrow, 0)
    for k in range(13):
        pltpu.sync_copy(r3, acc.at[pl.ds(s * 632 + k * 48, 48)])
    pltpu.sync_copy(r3.at[pl.ds(0, 8)], acc.at[pl.ds(s * 632 + 624, 8)])
    plsc.subcore_barrier()

    def _step(g0, _):
        for b in range(4):
            g = g0 * 4 + b
            bn = (b + 1) % 4

            @pl.when(g >= 3)
            def _():
                pltpu.make_async_copy(rows_v.at[bn], acc.at[dst_v.at[0]],
                                      sems[bn]).wait()
            gn = lax.rem(g + 1, CHUNKS)
            pltpu.make_async_copy(emb2.at[src_v.at[gn]], rows_v.at[bn],
                                  semg[bn]).start()
            pltpu.make_async_copy(emb2.at[src_v.at[g]], rows_v.at[b],
                                  semg[b]).wait()
            pltpu.async_copy(rows_v.at[b], acc.at[dst_v.at[g]], sems[b],
                             add=True)
        return 0
    lax.fori_loop(0, CHUNKS // 4, _step, 0)

    pltpu.make_async_copy(rows_v.at[1], acc.at[dst_v.at[0]], sems1).wait()
    pltpu.make_async_copy(rows_v.at[2], acc.at[dst_v.at[0]], sems2).wait()
    pltpu.make_async_copy(rows_v.at[3], acc.at[dst_v.at[0]], sems3).wait()
    pltpu.make_async_copy(emb2.at[src_v.at[0]], rows_v.at[0], semg0).wait()

    plsc.subcore_barrier()

    base = s * 624
    obase = c * N + base
    pltpu.sync_copy(acc.at[pl.ds(base, 624)], out_s.at[pl.ds(obase, 624)])

    @pl.when(s == NS - 1)
    def _tail():
        tb = NS * 624
        pltpu.sync_copy(acc.at[pl.ds(tb, 16)], out_s.at[pl.ds(c * N + tb, 16)])


def _sc_seg_body(tix, emb_flat, srcs, dsts, out_s,
                 src_v, dst_v, rows_v, acc,
                 semg0, semg1, semg2, semg3, sems0, sems1, sems2, sems3):
    c = lax.axis_index("c")
    s = lax.axis_index("s")
    _sc_run_dir(c, s, emb_flat.at[tix], srcs, dsts, out_s,
                src_v, dst_v, rows_v, acc,
                semg0, semg1, semg2, semg3, sems0, sems1, sems2, sems3)


def _make_sc_seg(tix):
    return pl.kernel(
        functools.partial(_sc_seg_body, tix),
        out_type=jax.ShapeDtypeStruct((NC * N, HALF), _f32),
        mesh=plsc.VectorSubcoreMesh(core_axis_name="c", subcore_axis_name="s",
                                    num_cores=NC, num_subcores=NS),
        compiler_params=pltpu.CompilerParams(use_tc_tiling_on_sc=False,
                                             skip_device_barrier=True),
        cost_estimate=pl.CostEstimate(flops=int(4.2e7),
                                      bytes_accessed=int(1.9e8),
                                      transcendentals=0),
        scratch_types=(
            pltpu.VMEM((CHUNKS, K), _i32),
            pltpu.VMEM((CHUNKS, K), _i32),
            pltpu.VMEM((4, K, HALF), _f32),
            pltpu.VMEM_SHARED((NROWS, HALF), _f32),
            pltpu.SemaphoreType.DMA,
            pltpu.SemaphoreType.DMA,
            pltpu.SemaphoreType.DMA,
            pltpu.SemaphoreType.DMA,
            pltpu.SemaphoreType.DMA,
            pltpu.SemaphoreType.DMA,
            pltpu.SemaphoreType.DMA,
            pltpu.SemaphoreType.DMA,
        ),
    )


_sc_seg_from_user = _make_sc_seg(0)
_sc_seg_from_item = _make_sc_seg(1)


def _sc_cnt_body(dsts2, out_c, dst_v, ones_v, zc, cnt_sp):
    c = lax.axis_index("c")
    s = lax.axis_index("s")
    w = c * NS + s
    pltpu.sync_copy(dsts2.at[w], dst_v)

    def _one(i, _):
        ones_v[pl.ds(i * 16, 16)] = jnp.ones((16,), _f32)
        return 0
    lax.fori_loop(0, KC // 16, _one, 0)

    def _z(i, _):
        zc[pl.ds(i * 16, 16)] = jnp.zeros((16,), _f32)
        return 0
    lax.fori_loop(0, 632 // 8, _z, 0)

    pltpu.sync_copy(zc.at[pl.ds(0, 632)], cnt_sp.at[pl.ds(s * 632, 632)])
    plsc.subcore_barrier()

    def _step(g, _):
        pltpu.sync_copy(ones_v, cnt_sp.at[dst_v.at[g]], add=True)
        return 0
    lax.fori_loop(0, CHUNKS_C, _step, 0)
    plsc.subcore_barrier()

    base = s * 624
    pltpu.sync_copy(cnt_sp.at[pl.ds(base, 624)], out_c.at[pl.ds(c * N + base, 624)])

    @pl.when(s == NS - 1)
    def _tail():
        tb = NS * 624
        pltpu.sync_copy(cnt_sp.at[pl.ds(tb, 16)], out_c.at[pl.ds(c * N + tb, 16)])


_sc_cnt = pl.kernel(
    _sc_cnt_body,
    out_type=jax.ShapeDtypeStruct((NC * N,), _f32),
    mesh=plsc.VectorSubcoreMesh(core_axis_name="c", subcore_axis_name="s",
                                num_cores=NC, num_subcores=NS),
    compiler_params=pltpu.CompilerParams(use_tc_tiling_on_sc=False,
                                         skip_device_barrier=True),
    cost_estimate=pl.CostEstimate(flops=int(3.3e5),
                                  bytes_accessed=int(3.0e6),
                                  transcendentals=0),
    scratch_types=(
        pltpu.VMEM((CHUNKS_C, KC), _i32),
        pltpu.VMEM((KC,), _f32),
        pltpu.VMEM((632,), _f32),
        pltpu.VMEM_SHARED((NROWS,), _f32),
    ),
)


def _edge_plan(edge_index):
    src = edge_index[0].astype(_i32)
    dst = edge_index[1].astype(_i32)
    pad = E_PAD - E
    r = jnp.arange(pad, dtype=_i32)
    src_p = jnp.concatenate([src, r % N])
    dst_p = jnp.concatenate([dst, N + (r % 16)])
    srcs = jnp.stack([src_p, src_p + N]).reshape(NC * NS, CHUNKS, K)
    dsts = dst_p.reshape(NS, CHUNKS, K)
    dsts_c = dst_p.reshape(NS, CHUNKS_C, KC)
    return srcs, dsts, dsts_c


_RB = 2000
_GRID = N // _RB


def _proj_body(xu_ref, xi_ref, w_ref, b_ref, out_ref):
    t = pl.program_id(0)
    x = jnp.where(t == 0, xu_ref[...], xi_ref[...])
    emb = jax.nn.relu(
        jnp.dot(x, w_ref[0], preferred_element_type=_f32) + b_ref[0])
    out_ref[0, 0] = emb[:, :HALF]
    out_ref[0, 1] = emb[:, HALF:]


_proj = pl.pallas_call(
    _proj_body,
    grid=(2, _GRID),
    in_specs=[
        pl.BlockSpec((_RB, D), lambda t, i: ((1 - t) * i, 0)),
        pl.BlockSpec((_RB, D), lambda t, i: (t * i, 0)),
        pl.BlockSpec((1, D, D), lambda t, i: (t, 0, 0)),
        pl.BlockSpec((1, 1, D), lambda t, i: (t, 0, 0)),
    ],
    out_specs=pl.BlockSpec((1, NC, _RB, HALF), lambda t, i: (t, 0, i, 0)),
    out_shape=jax.ShapeDtypeStruct((2, NC, N, HALF), _f32),
)


def _mixnorm_body(sums_ref, cnt_ref, emb_ref, wl_ref, bl_ref, wr_ref,
                  gw_ref, gb_ref, gms_ref, w2_ref, b2_ref,
                  out_ref, z_scr, st_scr):
    i = pl.program_id(0)

    @pl.when(i == 0)
    def _():
        st_scr[...] = jnp.zeros_like(st_scr)

    @pl.when(i < _GRID)
    def _():
        cnt = jnp.maximum(cnt_ref[...], 1.0)
        agg = jnp.concatenate([sums_ref[0], sums_ref[1]], axis=1) / cnt
        emb = jnp.concatenate([emb_ref[0, 0], emb_ref[0, 1]], axis=1)
        z = (jnp.dot(agg, wl_ref[...], preferred_element_type=_f32)
             + bl_ref[...]
             + jnp.dot(emb, wr_ref[...], preferred_element_type=_f32))
        z_scr[pl.ds(i, 1)] = z[None]
        st_scr[0:1, :] += jnp.sum(z, axis=0, keepdims=True)
        st_scr[1:2, :] += jnp.sum(z * z, axis=0, keepdims=True)

    @pl.when(i >= _GRID)
    def _():
        j = i - _GRID
        mean = st_scr[0:1, :] * (1.0 / N)
        msq = st_scr[1:2, :] * (1.0 / N)
        mm = gms_ref[...] * mean
        var = msq - 2.0 * mm * mean + mm * mm
        inv = lax.rsqrt(var + EPS)
        sc = gw_ref[...] * inv
        sh = gb_ref[...] - mm * sc
        zt = z_scr[j] * sc + sh
        out_ref[...] = (jnp.dot(zt, w2_ref[...], preferred_element_type=_f32)
                        + b2_ref[...])


def _make_mixnorm(T):
    clamp = lambda i: jnp.minimum(i, _GRID - 1)
    return pl.pallas_call(
        _mixnorm_body,
        grid=(2 * _GRID,),
        in_specs=[
            pl.BlockSpec((NC, _RB, HALF), lambda i: (0, clamp(i), 0)),
            pl.BlockSpec((_RB, 1), lambda i: ((1 - T) * _GRID + clamp(i), 0)),
            pl.BlockSpec((1, NC, _RB, HALF), lambda i: (T, 0, clamp(i), 0)),
            pl.BlockSpec((D, D), lambda i: (0, 0)),
            pl.BlockSpec((1, D), lambda i: (0, 0)),
            pl.BlockSpec((D, D), lambda i: (0, 0)),
            pl.BlockSpec((1, D), lambda i: (0, 0)),
            pl.BlockSpec((1, D), lambda i: (0, 0)),
            pl.BlockSpec((1, D), lambda i: (0, 0)),
            pl.BlockSpec((D, D), lambda i: (0, 0)),
            pl.BlockSpec((1, D), lambda i: (0, 0)),
        ],
        out_specs=pl.BlockSpec((_RB, D),
                               lambda i: (jnp.maximum(i - _GRID, 0), 0)),
        out_shape=jax.ShapeDtypeStruct((N, D), _f32),
        scratch_shapes=[
            pltpu.VMEM((_GRID, _RB, D), _f32),
            pltpu.VMEM((8, D), _f32),
        ],
    )


_mixnorm_item = _make_mixnorm(1)
_mixnorm_user = _make_mixnorm(0)


def kernel(x_user, x_item, edge_index_user_to_item, edge_index_item_to_user,
           W1_user, b1_user, W1_item, b1_item,
           Wl_ui, bl_ui, Wr_ui, Wl_iu, bl_iu, Wr_iu,
           gn_w_user, gn_b_user, gn_ms_user, gn_w_item, gn_b_item, gn_ms_item,
           W2_user, b2_user, W2_item, b2_item):
    row = lambda v: v.reshape(1, D)

    emb_all = _proj(x_user, x_item,
                    jnp.stack([W1_user, W1_item]),
                    jnp.stack([b1_user.reshape(1, D), b1_item.reshape(1, D)]))
    emb_flat = emb_all.reshape(2, NC * N, HALF)

    srcs_ui, dsts_ui, dsts_c_ui = _edge_plan(edge_index_user_to_item)
    srcs_iu, dsts_iu, dsts_c_iu = _edge_plan(edge_index_item_to_user)

    cnt2 = _sc_cnt(jnp.concatenate([dsts_c_ui, dsts_c_iu]).reshape(
        NC * NS, CHUNKS_C, KC)).reshape(NC * N, 1)
    sums_item = _sc_seg_from_user(emb_flat, srcs_ui, dsts_ui)
    sums_user = _sc_seg_from_item(emb_flat, srcs_iu, dsts_iu)

    out_item = _mixnorm_item(sums_item.reshape(NC, N, HALF), cnt2, emb_all,
                             Wl_ui, row(bl_ui), Wr_ui,
                             row(gn_w_item), row(gn_b_item), row(gn_ms_item),
                             W2_item, row(b2_item))
    out_user = _mixnorm_user(sums_user.reshape(NC, N, HALF), cnt2, emb_all,
                             Wl_iu, row(bl_iu), Wr_iu,
                             row(gn_w_user), row(gn_b_user), row(gn_ms_user),
                             W2_user, row(b2_user))
    return (out_user, out_item)


# speedup vs baseline: 1.2079x; 1.2079x over previous
import functools

import jax
import jax.numpy as jnp
from jax import lax
from jax.experimental import pallas as pl
from jax.experimental.pallas import tpu as pltpu
from jax.experimental.pallas import tpu_sc as plsc

N = 10000
E = 160000
D = 256
HALF = 128
EPS = 1e-5

NC, NS = 2, 16
K = 72
CHUNKS = 144
KC = 128
CHUNKS_C = 81
EP = K * CHUNKS
E_PAD = EP * NS
NROWS = 10112

_f32 = jnp.float32
_i32 = jnp.int32


def _sc_run_dir(c, s, emb2, srcs, dsts, out_s,
                src_v, dst_v, rows_v, acc,
                semg0, semg1, semg2, sems0, sems1, sems2):
    w = c * NS + s
    semg = (semg0, semg1, semg2)
    sems = (sems0, sems1, sems2)

    pltpu.sync_copy(srcs.at[w], src_v)
    pltpu.sync_copy(dsts.at[s], dst_v)

    pltpu.make_async_copy(emb2.at[src_v.at[0]], rows_v.at[0], semg0).start()

    r2 = rows_v.at[2]

    def _zrow(i, _):
        for j in range(8):
            rows_v[2, i, pl.ds(j * 16, 16)] = jnp.zeros((16,), _f32)
        return 0
    lax.fori_loop(0, K, _zrow, 0)
    for k in range(8):
        pltpu.sync_copy(r2, acc.at[pl.ds(s * 632 + k * 72, 72)])
    pltpu.sync_copy(r2.at[pl.ds(0, 56)], acc.at[pl.ds(s * 632 + 576, 56)])
    plsc.subcore_barrier()

    def _step(g0, _):
        for b in range(3):
            g = g0 * 3 + b
            bn = (b + 1) % 3

            @pl.when(g >= 2)
            def _():
                pltpu.make_async_copy(rows_v.at[bn], acc.at[dst_v.at[0]],
                                      sems[bn]).wait()
            gn = lax.rem(g + 1, CHUNKS)
            pltpu.make_async_copy(emb2.at[src_v.at[gn]], rows_v.at[bn],
                                  semg[bn]).start()
            pltpu.make_async_copy(emb2.at[src_v.at[g]], rows_v.at[b],
                                  semg[b]).wait()
            pltpu.async_copy(rows_v.at[b], acc.at[dst_v.at[g]], sems[b],
                             add=True)
        return 0
    lax.fori_loop(0, CHUNKS // 3, _step, 0)

    pltpu.make_async_copy(rows_v.at[1], acc.at[dst_v.at[0]], sems1).wait()
    pltpu.make_async_copy(rows_v.at[2], acc.at[dst_v.at[0]], sems2).wait()
    pltpu.make_async_copy(emb2.at[src_v.at[0]], rows_v.at[0], semg0).wait()

    plsc.subcore_barrier()

    base = s * 624
    obase = c * N + base
    pltpu.sync_copy(acc.at[pl.ds(base, 624)], out_s.at[pl.ds(obase, 624)])

    @pl.when(s == NS - 1)
    def _tail():
        tb = NS * 624
        pltpu.sync_copy(acc.at[pl.ds(tb, 16)], out_s.at[pl.ds(c * N + tb, 16)])


def _sc_seg_body(tix, emb_flat, srcs, dsts, out_s,
                 src_v, dst_v, rows_v, acc,
                 semg0, semg1, semg2, sems0, sems1, sems2):
    c = lax.axis_index("c")
    s = lax.axis_index("s")
    _sc_run_dir(c, s, emb_flat.at[tix], srcs, dsts, out_s,
                src_v, dst_v, rows_v, acc,
                semg0, semg1, semg2, sems0, sems1, sems2)


def _make_sc_seg(tix):
    return pl.kernel(
        functools.partial(_sc_seg_body, tix),
        out_type=jax.ShapeDtypeStruct((NC * N, HALF), _f32),
        mesh=plsc.VectorSubcoreMesh(core_axis_name="c", subcore_axis_name="s",
                                    num_cores=NC, num_subcores=NS),
        compiler_params=pltpu.CompilerParams(use_tc_tiling_on_sc=False,
                                             skip_device_barrier=True),
        cost_estimate=pl.CostEstimate(flops=int(4.2e7),
                                      bytes_accessed=int(1.9e8),
                                      transcendentals=0),
        scratch_types=(
            pltpu.VMEM((CHUNKS, K), _i32),
            pltpu.VMEM((CHUNKS, K), _i32),
            pltpu.VMEM((3, K, HALF), _f32),
            pltpu.VMEM_SHARED((NROWS, HALF), _f32),
            pltpu.SemaphoreType.DMA,
            pltpu.SemaphoreType.DMA,
            pltpu.SemaphoreType.DMA,
            pltpu.SemaphoreType.DMA,
            pltpu.SemaphoreType.DMA,
            pltpu.SemaphoreType.DMA,
        ),
    )


_sc_seg_from_user = _make_sc_seg(0)
_sc_seg_from_item = _make_sc_seg(1)


def _sc_cnt_body(dsts2, out_c, dst_v, ones_v, zc, cnt_sp):
    c = lax.axis_index("c")
    s = lax.axis_index("s")
    w = c * NS + s
    pltpu.sync_copy(dsts2.at[w], dst_v)

    def _one(i, _):
        ones_v[pl.ds(i * 16, 16)] = jnp.ones((16,), _f32)
        return 0
    lax.fori_loop(0, KC // 16, _one, 0)

    def _z(i, _):
        zc[pl.ds(i * 16, 16)] = jnp.zeros((16,), _f32)
        return 0
    lax.fori_loop(0, 632 // 8, _z, 0)

    pltpu.sync_copy(zc.at[pl.ds(0, 632)], cnt_sp.at[pl.ds(s * 632, 632)])
    plsc.subcore_barrier()

    def _step(g, _):
        pltpu.sync_copy(ones_v, cnt_sp.at[dst_v.at[g]], add=True)
        return 0
    lax.fori_loop(0, CHUNKS_C, _step, 0)
    plsc.subcore_barrier()

    base = s * 624
    pltpu.sync_copy(cnt_sp.at[pl.ds(base, 624)], out_c.at[pl.ds(c * N + base, 624)])

    @pl.when(s == NS - 1)
    def _tail():
        tb = NS * 624
        pltpu.sync_copy(cnt_sp.at[pl.ds(tb, 16)], out_c.at[pl.ds(c * N + tb, 16)])


_sc_cnt = pl.kernel(
    _sc_cnt_body,
    out_type=jax.ShapeDtypeStruct((NC * N,), _f32),
    mesh=plsc.VectorSubcoreMesh(core_axis_name="c", subcore_axis_name="s",
                                num_cores=NC, num_subcores=NS),
    compiler_params=pltpu.CompilerParams(use_tc_tiling_on_sc=False,
                                         skip_device_barrier=True),
    cost_estimate=pl.CostEstimate(flops=int(3.3e5),
                                  bytes_accessed=int(3.0e6),
                                  transcendentals=0),
    scratch_types=(
        pltpu.VMEM((CHUNKS_C, KC), _i32),
        pltpu.VMEM((KC,), _f32),
        pltpu.VMEM((632,), _f32),
        pltpu.VMEM_SHARED((NROWS,), _f32),
    ),
)


def _edge_plan(edge_index):
    src = edge_index[0].astype(_i32)
    dst = edge_index[1].astype(_i32)
    pad = E_PAD - E
    r = jnp.arange(pad, dtype=_i32)
    src_p = jnp.concatenate([src, r % N])
    dst_p = jnp.concatenate([dst, N + (r % 16)])
    srcs = jnp.stack([src_p, src_p + N]).reshape(NC * NS, CHUNKS, K)
    dsts = dst_p.reshape(NS, CHUNKS, K)
    dsts_c = dst_p.reshape(NS, CHUNKS_C, KC)
    return srcs, dsts, dsts_c


_RB = 2000
_GRID = N // _RB


def _proj_body(xu_ref, xi_ref, w_ref, b_ref, out_ref):
    t = pl.program_id(0)
    x = jnp.where(t == 0, xu_ref[...], xi_ref[...])
    emb = jax.nn.relu(
        jnp.dot(x, w_ref[0], preferred_element_type=_f32) + b_ref[0])
    out_ref[0, 0] = emb[:, :HALF]
    out_ref[0, 1] = emb[:, HALF:]


_proj = pl.pallas_call(
    _proj_body,
    grid=(2, _GRID),
    in_specs=[
        pl.BlockSpec((_RB, D), lambda t, i: ((1 - t) * i, 0)),
        pl.BlockSpec((_RB, D), lambda t, i: (t * i, 0)),
        pl.BlockSpec((1, D, D), lambda t, i: (t, 0, 0)),
        pl.BlockSpec((1, 1, D), lambda t, i: (t, 0, 0)),
    ],
    out_specs=pl.BlockSpec((1, NC, _RB, HALF), lambda t, i: (t, 0, i, 0)),
    out_shape=jax.ShapeDtypeStruct((2, NC, N, HALF), _f32),
)


def _mixnorm_body(sums_ref, cnt_ref, emb_ref, wl_ref, bl_ref, wr_ref,
                  gw_ref, gb_ref, gms_ref, w2_ref, b2_ref,
                  out_ref, z_scr, st_scr):
    i = pl.program_id(0)

    @pl.when(i == 0)
    def _():
        st_scr[...] = jnp.zeros_like(st_scr)

    @pl.when(i < _GRID)
    def _():
        cnt = jnp.maximum(cnt_ref[...], 1.0)
        agg = jnp.concatenate([sums_ref[0], sums_ref[1]], axis=1) / cnt
        emb = jnp.concatenate([emb_ref[0, 0], emb_ref[0, 1]], axis=1)
        z = (jnp.dot(agg, wl_ref[...], preferred_element_type=_f32)
             + bl_ref[...]
             + jnp.dot(emb, wr_ref[...], preferred_element_type=_f32))
        z_scr[pl.ds(i, 1)] = z[None]
        st_scr[0:1, :] += jnp.sum(z, axis=0, keepdims=True)
        st_scr[1:2, :] += jnp.sum(z * z, axis=0, keepdims=True)

    @pl.when(i >= _GRID)
    def _():
        j = i - _GRID
        mean = st_scr[0:1, :] * (1.0 / N)
        msq = st_scr[1:2, :] * (1.0 / N)
        mm = gms_ref[...] * mean
        var = msq - 2.0 * mm * mean + mm * mm
        inv = lax.rsqrt(var + EPS)
        sc = gw_ref[...] * inv
        sh = gb_ref[...] - mm * sc
        zt = z_scr[j] * sc + sh
        out_ref[...] = (jnp.dot(zt, w2_ref[...], preferred_element_type=_f32)
                        + b2_ref[...])


def _make_mixnorm(T):
    clamp = lambda i: jnp.minimum(i, _GRID - 1)
    return pl.pallas_call(
        _mixnorm_body,
        grid=(2 * _GRID,),
        in_specs=[
            pl.BlockSpec((NC, _RB, HALF), lambda i: (0, clamp(i), 0)),
            pl.BlockSpec((_RB, 1), lambda i: ((1 - T) * _GRID + clamp(i), 0)),
            pl.BlockSpec((1, NC, _RB, HALF), lambda i: (T, 0, clamp(i), 0)),
            pl.BlockSpec((D, D), lambda i: (0, 0)),
            pl.BlockSpec((1, D), lambda i: (0, 0)),
            pl.BlockSpec((D, D), lambda i: (0, 0)),
            pl.BlockSpec((1, D), lambda i: (0, 0)),
            pl.BlockSpec((1, D), lambda i: (0, 0)),
            pl.BlockSpec((1, D), lambda i: (0, 0)),
            pl.BlockSpec((D, D), lambda i: (0, 0)),
            pl.BlockSpec((1, D), lambda i: (0, 0)),
        ],
        out_specs=pl.BlockSpec((_RB, D),
                               lambda i: (jnp.maximum(i - _GRID, 0), 0)),
        out_shape=jax.ShapeDtypeStruct((N, D), _f32),
        scratch_shapes=[
            pltpu.VMEM((_GRID, _RB, D), _f32),
            pltpu.VMEM((8, D), _f32),
        ],
    )


_mixnorm_item = _make_mixnorm(1)
_mixnorm_user = _make_mixnorm(0)


def kernel(x_user, x_item, edge_index_user_to_item, edge_index_item_to_user,
           W1_user, b1_user, W1_item, b1_item,
           Wl_ui, bl_ui, Wr_ui, Wl_iu, bl_iu, Wr_iu,
           gn_w_user, gn_b_user, gn_ms_user, gn_w_item, gn_b_item, gn_ms_item,
           W2_user, b2_user, W2_item, b2_item):
    row = lambda v: v.reshape(1, D)

    emb_all = _proj(x_user, x_item,
                    jnp.stack([W1_user, W1_item]),
                    jnp.stack([b1_user.reshape(1, D), b1_item.reshape(1, D)]))
    emb_flat = emb_all.reshape(2, NC * N, HALF)

    srcs_ui, dsts_ui, dsts_c_ui = _edge_plan(edge_index_user_to_item)
    srcs_iu, dsts_iu, dsts_c_iu = _edge_plan(edge_index_item_to_user)

    cnt2 = _sc_cnt(jnp.concatenate([dsts_c_ui, dsts_c_iu]).reshape(
        NC * NS, CHUNKS_C, KC)).reshape(NC * N, 1)
    sums_item = _sc_seg_from_user(emb_flat, srcs_ui, dsts_ui)
    sums_user = _sc_seg_from_item(emb_flat, srcs_iu, dsts_iu)

    out_item = _mixnorm_item(sums_item.reshape(NC, N, HALF), cnt2, emb_all,
                             Wl_ui, row(bl_ui), Wr_ui,
                             row(gn_w_item), row(gn_b_item), row(gn_ms_item),
                             W2_item, row(b2_item))
    out_user = _mixnorm_user(sums_user.reshape(NC, N, HALF), cnt2, emb_all,
                             Wl_iu, row(bl_iu), Wr_iu,
                             row(gn_w_user), row(gn_b_user), row(gn_ms_user),
                             W2_user, row(b2_user))
    return (out_user, out_item)


# speedup vs baseline: 1.2080x; 1.0000x over previous
import functools

import jax
import jax.numpy as jnp
from jax import lax
from jax.experimental import pallas as pl
from jax.experimental.pallas import tpu as pltpu
from jax.experimental.pallas import tpu_sc as plsc

N = 10000
E = 160000
D = 256
HALF = 128
EPS = 1e-5

NC, NS = 2, 16
K = 72
CHUNKS = 144
KC = 128
CHUNKS_C = 81
EP = K * CHUNKS
E_PAD = EP * NS
NROWS = 10112

_f32 = jnp.float32
_i32 = jnp.int32


def _sc_run_dir(c, s, emb2, srcs, dsts, out_s,
                src_v, dst_v, rows_v, acc,
                semg0, semg1, semg2, sems0, sems1, sems2):
    w = c * NS + s
    semg = (semg0, semg1, semg2)
    sems = (sems0, sems1, sems2)

    pltpu.sync_copy(srcs.at[w], src_v)
    pltpu.sync_copy(dsts.at[s], dst_v)

    pltpu.make_async_copy(emb2.at[src_v.at[0]], rows_v.at[0], semg0).start()

    r2 = rows_v.at[2]

    def _zrow(i, _):
        for j in range(8):
            rows_v[2, i, pl.ds(j * 16, 16)] = jnp.zeros((16,), _f32)
        return 0
    lax.fori_loop(0, K, _zrow, 0)
    for k in range(8):
        pltpu.sync_copy(r2, acc.at[pl.ds(s * 632 + k * 72, 72)])
    pltpu.sync_copy(r2.at[pl.ds(0, 56)], acc.at[pl.ds(s * 632 + 576, 56)])
    plsc.subcore_barrier()

    def _step(g0, _):
        for b in range(3):
            g = g0 * 3 + b
            bn = (b + 1) % 3

            @pl.when(g >= 2)
            def _():
                pltpu.make_async_copy(rows_v.at[bn], acc.at[dst_v.at[0]],
                                      sems[bn]).wait()
            gn = lax.rem(g + 1, CHUNKS)
            pltpu.make_async_copy(emb2.at[src_v.at[gn]], rows_v.at[bn],
                                  semg[bn]).start()
            pltpu.make_async_copy(emb2.at[src_v.at[g]], rows_v.at[b],
                                  semg[b]).wait()
            pltpu.async_copy(rows_v.at[b], acc.at[dst_v.at[g]], sems[b],
                             add=True)
        return 0
    lax.fori_loop(0, CHUNKS // 3, _step, 0)

    pltpu.make_async_copy(rows_v.at[1], acc.at[dst_v.at[0]], sems1).wait()
    pltpu.make_async_copy(rows_v.at[2], acc.at[dst_v.at[0]], sems2).wait()
    pltpu.make_async_copy(emb2.at[src_v.at[0]], rows_v.at[0], semg0).wait()

    plsc.subcore_barrier()

    base = s * 624
    obase = c * N + base
    pltpu.sync_copy(acc.at[pl.ds(base, 624)], out_s.at[pl.ds(obase, 624)])

    @pl.when(s == NS - 1)
    def _tail():
        tb = NS * 624
        pltpu.sync_copy(acc.at[pl.ds(tb, 16)], out_s.at[pl.ds(c * N + tb, 16)])


def _sc_seg_body(tix, emb_flat, srcs, dsts, out_s,
                 src_v, dst_v, rows_v, acc,
                 semg0, semg1, semg2, sems0, sems1, sems2):
    c = lax.axis_index("c")
    s = lax.axis_index("s")
    _sc_run_dir(c, s, emb_flat.at[tix], srcs, dsts, out_s,
                src_v, dst_v, rows_v, acc,
                semg0, semg1, semg2, sems0, sems1, sems2)


def _make_sc_seg(tix):
    return pl.kernel(
        functools.partial(_sc_seg_body, tix),
        out_type=jax.ShapeDtypeStruct((NC * N, HALF), _f32),
        mesh=plsc.VectorSubcoreMesh(core_axis_name="c", subcore_axis_name="s",
                                    num_cores=NC, num_subcores=NS),
        compiler_params=pltpu.CompilerParams(use_tc_tiling_on_sc=False),
        cost_estimate=pl.CostEstimate(flops=int(4.2e7),
                                      bytes_accessed=int(1.9e8),
                                      transcendentals=0),
        scratch_types=(
            pltpu.VMEM((CHUNKS, K), _i32),
            pltpu.VMEM((CHUNKS, K), _i32),
            pltpu.VMEM((3, K, HALF), _f32),
            pltpu.VMEM_SHARED((NROWS, HALF), _f32),
            pltpu.SemaphoreType.DMA,
            pltpu.SemaphoreType.DMA,
            pltpu.SemaphoreType.DMA,
            pltpu.SemaphoreType.DMA,
            pltpu.SemaphoreType.DMA,
            pltpu.SemaphoreType.DMA,
        ),
    )


_sc_seg_from_user = _make_sc_seg(0)
_sc_seg_from_item = _make_sc_seg(1)


def _sc_cnt_body(dsts2, out_c, dst_v, ones_v, zc, cnt_sp):
    c = lax.axis_index("c")
    s = lax.axis_index("s")
    w = c * NS + s
    pltpu.sync_copy(dsts2.at[w], dst_v)

    def _one(i, _):
        ones_v[pl.ds(i * 16, 16)] = jnp.ones((16,), _f32)
        return 0
    lax.fori_loop(0, KC // 16, _one, 0)

    def _z(i, _):
        zc[pl.ds(i * 16, 16)] = jnp.zeros((16,), _f32)
        return 0
    lax.fori_loop(0, 632 // 8, _z, 0)

    pltpu.sync_copy(zc.at[pl.ds(0, 632)], cnt_sp.at[pl.ds(s * 632, 632)])
    plsc.subcore_barrier()

    def _step(g, _):
        pltpu.sync_copy(ones_v, cnt_sp.at[dst_v.at[g]], add=True)
        return 0
    lax.fori_loop(0, CHUNKS_C, _step, 0)
    plsc.subcore_barrier()

    base = s * 624
    pltpu.sync_copy(cnt_sp.at[pl.ds(base, 624)], out_c.at[pl.ds(c * N + base, 624)])

    @pl.when(s == NS - 1)
    def _tail():
        tb = NS * 624
        pltpu.sync_copy(cnt_sp.at[pl.ds(tb, 16)], out_c.at[pl.ds(c * N + tb, 16)])


_sc_cnt = pl.kernel(
    _sc_cnt_body,
    out_type=jax.ShapeDtypeStruct((NC * N,), _f32),
    mesh=plsc.VectorSubcoreMesh(core_axis_name="c", subcore_axis_name="s",
                                num_cores=NC, num_subcores=NS),
    compiler_params=pltpu.CompilerParams(use_tc_tiling_on_sc=False),
    cost_estimate=pl.CostEstimate(flops=int(3.3e5),
                                  bytes_accessed=int(3.0e6),
                                  transcendentals=0),
    scratch_types=(
        pltpu.VMEM((CHUNKS_C, KC), _i32),
        pltpu.VMEM((KC,), _f32),
        pltpu.VMEM((632,), _f32),
        pltpu.VMEM_SHARED((NROWS,), _f32),
    ),
)


def _edge_plan(edge_index):
    src = edge_index[0].astype(_i32)
    dst = edge_index[1].astype(_i32)
    pad = E_PAD - E
    r = jnp.arange(pad, dtype=_i32)
    src_p = jnp.concatenate([src, r % N])
    dst_p = jnp.concatenate([dst, N + (r % 16)])
    srcs = jnp.stack([src_p, src_p + N]).reshape(NC * NS, CHUNKS, K)
    dsts = dst_p.reshape(NS, CHUNKS, K)
    dsts_c = dst_p.reshape(NS, CHUNKS_C, KC)
    return srcs, dsts, dsts_c


_RB = 2000
_GRID = N // _RB


def _proj_body(xu_ref, xi_ref, w_ref, b_ref, out_ref):
    t = pl.program_id(0)
    x = jnp.where(t == 0, xu_ref[...], xi_ref[...])
    emb = jax.nn.relu(
        jnp.dot(x, w_ref[0], preferred_element_type=_f32) + b_ref[0])
    out_ref[0, 0] = emb[:, :HALF]
    out_ref[0, 1] = emb[:, HALF:]


_proj = pl.pallas_call(
    _proj_body,
    grid=(2, _GRID),
    in_specs=[
        pl.BlockSpec((_RB, D), lambda t, i: ((1 - t) * i, 0)),
        pl.BlockSpec((_RB, D), lambda t, i: (t * i, 0)),
        pl.BlockSpec((1, D, D), lambda t, i: (t, 0, 0)),
        pl.BlockSpec((1, 1, D), lambda t, i: (t, 0, 0)),
    ],
    out_specs=pl.BlockSpec((1, NC, _RB, HALF), lambda t, i: (t, 0, i, 0)),
    out_shape=jax.ShapeDtypeStruct((2, NC, N, HALF), _f32),
)


def _mixnorm_body(sums_ref, cnt_ref, emb_ref, wl_ref, bl_ref, wr_ref,
                  gw_ref, gb_ref, gms_ref, w2_ref, b2_ref,
                  out_ref, z_scr, st_scr):
    i = pl.program_id(0)

    @pl.when(i == 0)
    def _():
        st_scr[...] = jnp.zeros_like(st_scr)

    @pl.when(i < _GRID)
    def _():
        cnt = jnp.maximum(cnt_ref[...], 1.0)
        agg = jnp.concatenate([sums_ref[0], sums_ref[1]], axis=1) / cnt
        emb = jnp.concatenate([emb_ref[0, 0], emb_ref[0, 1]], axis=1)
        z = (jnp.dot(agg, wl_ref[...], preferred_element_type=_f32)
             + bl_ref[...]
             + jnp.dot(emb, wr_ref[...], preferred_element_type=_f32))
        z_scr[pl.ds(i, 1)] = z[None]
        st_scr[0:1, :] += jnp.sum(z, axis=0, keepdims=True)
        st_scr[1:2, :] += jnp.sum(z * z, axis=0, keepdims=True)

    @pl.when(i >= _GRID)
    def _():
        j = i - _GRID
        mean = st_scr[0:1, :] * (1.0 / N)
        msq = st_scr[1:2, :] * (1.0 / N)
        mm = gms_ref[...] * mean
        var = msq - 2.0 * mm * mean + mm * mm
        inv = lax.rsqrt(var + EPS)
        sc = gw_ref[...] * inv
        sh = gb_ref[...] - mm * sc
        zt = z_scr[j] * sc + sh
        out_ref[...] = (jnp.dot(zt, w2_ref[...], preferred_element_type=_f32)
                        + b2_ref[...])


def _make_mixnorm(T):
    clamp = lambda i: jnp.minimum(i, _GRID - 1)
    return pl.pallas_call(
        _mixnorm_body,
        grid=(2 * _GRID,),
        in_specs=[
            pl.BlockSpec((NC, _RB, HALF), lambda i: (0, clamp(i), 0)),
            pl.BlockSpec((_RB, 1), lambda i: ((1 - T) * _GRID + clamp(i), 0)),
            pl.BlockSpec((1, NC, _RB, HALF), lambda i: (T, 0, clamp(i), 0)),
            pl.BlockSpec((D, D), lambda i: (0, 0)),
            pl.BlockSpec((1, D), lambda i: (0, 0)),
            pl.BlockSpec((D, D), lambda i: (0, 0)),
            pl.BlockSpec((1, D), lambda i: (0, 0)),
            pl.BlockSpec((1, D), lambda i: (0, 0)),
            pl.BlockSpec((1, D), lambda i: (0, 0)),
            pl.BlockSpec((D, D), lambda i: (0, 0)),
            pl.BlockSpec((1, D), lambda i: (0, 0)),
        ],
        out_specs=pl.BlockSpec((_RB, D),
                               lambda i: (jnp.maximum(i - _GRID, 0), 0)),
        out_shape=jax.ShapeDtypeStruct((N, D), _f32),
        scratch_shapes=[
            pltpu.VMEM((_GRID, _RB, D), _f32),
            pltpu.VMEM((8, D), _f32),
        ],
    )


_mixnorm_item = _make_mixnorm(1)
_mixnorm_user = _make_mixnorm(0)


def kernel(x_user, x_item, edge_index_user_to_item, edge_index_item_to_user,
           W1_user, b1_user, W1_item, b1_item,
           Wl_ui, bl_ui, Wr_ui, Wl_iu, bl_iu, Wr_iu,
           gn_w_user, gn_b_user, gn_ms_user, gn_w_item, gn_b_item, gn_ms_item,
           W2_user, b2_user, W2_item, b2_item):
    row = lambda v: v.reshape(1, D)

    emb_all = _proj(x_user, x_item,
                    jnp.stack([W1_user, W1_item]),
                    jnp.stack([b1_user.reshape(1, D), b1_item.reshape(1, D)]))
    emb_flat = emb_all.reshape(2, NC * N, HALF)

    srcs_ui, dsts_ui, dsts_c_ui = _edge_plan(edge_index_user_to_item)
    srcs_iu, dsts_iu, dsts_c_iu = _edge_plan(edge_index_item_to_user)

    cnt2 = _sc_cnt(jnp.concatenate([dsts_c_ui, dsts_c_iu]).reshape(
        NC * NS, CHUNKS_C, KC)).reshape(NC * N, 1)
    sums_item = _sc_seg_from_user(emb_flat, srcs_ui, dsts_ui)
    sums_user = _sc_seg_from_item(emb_flat, srcs_iu, dsts_iu)

    out_item = _mixnorm_item(sums_item.reshape(NC, N, HALF), cnt2, emb_all,
                             Wl_ui, row(bl_ui), Wr_ui,
                             row(gn_w_item), row(gn_b_item), row(gn_ms_item),
                             W2_item, row(b2_item))
    out_user = _mixnorm_user(sums_user.reshape(NC, N, HALF), cnt2, emb_all,
                             Wl_iu, row(bl_iu), Wr_iu,
                             row(gn_w_user), row(gn_b_user), row(gn_ms_user),
                             W2_user, row(b2_user))
    return (out_user, out_item)
